# Initial kernel scaffold; baseline (speedup 1.0000x reference)
import jax
import jax.numpy as jnp
from jax.experimental import pallas as pl


def kernel(x, edge_index, batch, W1, b1, W2, b2, Wfc, bfc):
    raise NotImplementedError("write your pallas kernel here")


# speedup vs baseline: 2.3279x; 2.3279x over previous
import jax
import jax.numpy as jnp
from jax.experimental import pallas as pl

N = 10000
G = 128
D_H = 512
D_OUT = 16


def _gcn_conv(x, edge_index, W, b, dinv):
    h = x @ W
    src = edge_index[0]
    dst = edge_index[1]
    hp = h * dinv[:, None]
    msg = hp[src]
    out = jnp.zeros_like(h).at[dst].add(msg)
    out = (out + hp) * dinv[:, None]
    return out + b


def _final_body(pooled_ref, cnt_ref, wfc_ref, bfc_ref, out_ref):
    cnt = jnp.maximum(cnt_ref[...], 1.0)
    pooled = pooled_ref[...] / cnt[:, None]
    logits = jnp.dot(pooled, wfc_ref[...], preferred_element_type=jnp.float32)
    logits = logits + bfc_ref[...][None, :]
    m = jnp.max(logits, axis=1, keepdims=True)
    e = jnp.exp(logits - m)
    lse = jnp.log(jnp.sum(e, axis=1, keepdims=True)) + m
    out_ref[...] = logits - lse


def kernel(x, edge_index, batch, W1, b1, W2, b2, Wfc, bfc):
    dst = edge_index[1]
    deg = jnp.zeros((N,), jnp.float32).at[dst].add(1.0) + 1.0
    dinv = jax.lax.rsqrt(deg)
    h = jax.nn.relu(_gcn_conv(x, edge_index, W1, b1, dinv))
    h = jax.nn.relu(_gcn_conv(h, edge_index, W2, b2, dinv))
    s = jax.ops.segment_sum(h, batch, num_segments=G)
    cnt = jax.ops.segment_sum(jnp.ones((N,), jnp.float32), batch, num_segments=G)
    out = pl.pallas_call(
        _final_body,
        out_shape=jax.ShapeDtypeStruct((G, D_OUT), jnp.float32),
    )(s, cnt, Wfc, bfc)
    return out


# speedup vs baseline: 4.1353x; 1.7764x over previous
import functools

import jax
import jax.numpy as jnp
from jax import lax
from jax.experimental import pallas as pl
from jax.experimental.pallas import tpu as pltpu
import jax.experimental.pallas.tpu_sc as plsc

N = 10000
E = 160000
G = 128
D_IN = 256
D_H = 512
D_OUT = 16

NC = 2
NS = 16
NW = NC * NS
L = 16

CW = 128
NCHUNK = D_H // CW
N_PAD = 10240
STRIPE = N_PAD // NS
EB = 128
NB = 40
E_PAD = NW * NB * EB
TILE_N = 1280
GRID_N = N_PAD // TILE_N


def _deg_body(dst_hbm, out_hbm, idx_buf, ones_buf, zstripe, acc_sh):
    core = lax.axis_index("c")
    sid = lax.axis_index("s")
    wid = sid * NC + core
    one16 = jnp.ones((L,), jnp.float32)
    zero16 = jnp.zeros((L,), jnp.float32)

    def fill_ones(t, _):
        ones_buf[pl.ds(t * L, L)] = one16
        return 0
    lax.fori_loop(0, EB // L, fill_ones, 0)

    def fill_z(t, _):
        zstripe[pl.ds(t * L, L)] = zero16
        return 0
    lax.fori_loop(0, STRIPE // L, fill_z, 0)

    pltpu.sync_copy(zstripe, acc_sh.at[pl.ds(sid * STRIPE, STRIPE)])
    pltpu.sync_copy(dst_hbm.at[pl.ds(wid * NB, NB), :], idx_buf)
    plsc.subcore_barrier()

    def add_batch(j, _):
        pltpu.sync_copy(ones_buf, acc_sh.at[idx_buf.at[j]], add=True)
        return 0
    lax.fori_loop(0, NB, add_batch, 0)
    plsc.subcore_barrier()

    pltpu.sync_copy(acc_sh.at[pl.ds(sid * STRIPE, STRIPE)],
                    out_hbm.at[pl.ds(core * N_PAD + sid * STRIPE, STRIPE)])


@functools.cache
def _deg_call():
    mesh = plsc.VectorSubcoreMesh(core_axis_name="c", subcore_axis_name="s",
                                  num_cores=NC, num_subcores=NS)
    return pl.kernel(
        _deg_body,
        out_type=jax.ShapeDtypeStruct((NC * N_PAD,), jnp.float32),
        mesh=mesh,
        scratch_types=[
            pltpu.VMEM((NB, EB), jnp.int32),
            pltpu.VMEM((EB,), jnp.float32),
            pltpu.VMEM((STRIPE,), jnp.float32),
            pltpu.VMEM_SHARED((N_PAD,), jnp.float32),
        ],
    )


def _spmm_body(mp_hbm, src_hbm, dst_hbm, out_hbm,
               src_buf, dst_buf, rows_a, rows_b, sem_a, sem_b, acc_sh):
    core = lax.axis_index("c")
    sid = lax.axis_index("s")
    wid = sid * NC + core
    zero16 = jnp.zeros((L,), jnp.float32)

    base = wid * NB
    pltpu.sync_copy(src_hbm.at[pl.ds(base, NB), :], src_buf)
    pltpu.sync_copy(dst_hbm.at[pl.ds(base, NB), :], dst_buf)

    for c in range(NCHUNK):
        def fill_z(t, _):
            r = t // (CW // L)
            k = (t % (CW // L)) * L
            rows_a[r, pl.ds(k, L)] = zero16
            return 0
        lax.fori_loop(0, EB * CW // L, fill_z, 0)
        for r in range(STRIPE // EB):
            pltpu.sync_copy(rows_a, acc_sh.at[pl.ds(sid * STRIPE + r * EB, EB), :])
        plsc.subcore_barrier()
        table = mp_hbm.at[c]
        pltpu.async_copy(table.at[src_buf.at[0]], rows_a, sem_a)
        pltpu.async_copy(table.at[src_buf.at[1]], rows_b, sem_b)

        def step(t, _):
            for b, (rbuf, sem) in enumerate(((rows_a, sem_a), (rows_b, sem_b))):
                j = 2 * t + b
                pltpu.make_async_copy(table.at[src_buf.at[j]], rbuf, sem).wait()
                pltpu.sync_copy(rbuf, acc_sh.at[dst_buf.at[j]], add=True)
                jj = j + 2

                @pl.when(jj < NB)
                def _():
                    pltpu.async_copy(table.at[src_buf.at[jj]], rbuf, sem)
            return 0
        lax.fori_loop(0, NB // 2, step, 0)
        plsc.subcore_barrier()
        pltpu.sync_copy(
            acc_sh.at[pl.ds(sid * STRIPE, STRIPE), :],
            out_hbm.at[c].at[pl.ds(core * N_PAD + sid * STRIPE, STRIPE), :])
        plsc.subcore_barrier()


@functools.cache
def _spmm_call():
    mesh = plsc.VectorSubcoreMesh(core_axis_name="c", subcore_axis_name="s",
                                  num_cores=NC, num_subcores=NS)
    return pl.kernel(
        _spmm_body,
        out_type=jax.ShapeDtypeStruct((NCHUNK, NC * N_PAD, CW), jnp.float32),
        mesh=mesh,
        scratch_types=[
            pltpu.VMEM((NB, EB), jnp.int32),
            pltpu.VMEM((NB, EB), jnp.int32),
            pltpu.VMEM((EB, CW), jnp.float32),
            pltpu.VMEM((EB, CW), jnp.float32),
            pltpu.SemaphoreType.DMA,
            pltpu.SemaphoreType.DMA,
            pltpu.VMEM_SHARED((N_PAD, CW), jnp.float32),
        ],
    )


def _mm1_body(x_ref, w_ref, dinv_ref, out_ref):
    m = jnp.dot(x_ref[...], w_ref[...], preferred_element_type=jnp.float32)
    m = m * dinv_ref[...]
    for c in range(NCHUNK):
        out_ref[c] = m[:, c * CW:(c + 1) * CW]


_mm1_call = pl.pallas_call(
    _mm1_body,
    grid=(GRID_N,),
    in_specs=[
        pl.BlockSpec((TILE_N, D_IN), lambda i: (i, 0)),
        pl.BlockSpec((D_IN, D_H), lambda i: (0, 0)),
        pl.BlockSpec((TILE_N, 1), lambda i: (i, 0)),
    ],
    out_specs=pl.BlockSpec((NCHUNK, TILE_N, CW), lambda i: (0, i, 0)),
    out_shape=jax.ShapeDtypeStruct((NCHUNK, N_PAD, CW), jnp.float32),
)


def _mm2_body(s_ref, m_ref, dinv_ref, b_ref, w_ref, out_ref):
    dinv = dinv_ref[...]
    acc = jnp.zeros((TILE_N, D_H), jnp.float32)
    for c in range(NCHUNK):
        mc = m_ref[c]
        sc = s_ref[c, 0] + s_ref[c, 1] + mc
        hc = jnp.maximum(sc * dinv + b_ref[:, c * CW:(c + 1) * CW], 0.0)
        acc = acc + jnp.dot(hc, w_ref[c * CW:(c + 1) * CW, :],
                            preferred_element_type=jnp.float32)
    acc = acc * dinv
    for c in range(NCHUNK):
        out_ref[c] = acc[:, c * CW:(c + 1) * CW]


_mm2_call = pl.pallas_call(
    _mm2_body,
    grid=(GRID_N,),
    in_specs=[
        pl.BlockSpec((NCHUNK, NC, TILE_N, CW), lambda i: (0, 0, i, 0)),
        pl.BlockSpec((NCHUNK, TILE_N, CW), lambda i: (0, i, 0)),
        pl.BlockSpec((TILE_N, 1), lambda i: (i, 0)),
        pl.BlockSpec((1, D_H), lambda i: (0, 0)),
        pl.BlockSpec((D_H, D_H), lambda i: (0, 0)),
    ],
    out_specs=pl.BlockSpec((NCHUNK, TILE_N, CW), lambda i: (0, i, 0)),
    out_shape=jax.ShapeDtypeStruct((NCHUNK, N_PAD, CW), jnp.float32),
)


def _fin_body(s_ref, m_ref, dinv_ref, b_ref, batch_ref, wfc_ref, bfc_ref,
              out_ref, pooled_acc, cnt_acc):
    i = pl.program_id(0)

    @pl.when(i == 0)
    def _():
        pooled_acc[...] = jnp.zeros_like(pooled_acc)
        cnt_acc[...] = jnp.zeros_like(cnt_acc)

    dinv = dinv_ref[...]
    bt = batch_ref[...]
    gi = lax.broadcasted_iota(jnp.int32, (G, TILE_N), 0)
    oh = jnp.where(gi == bt, 1.0, 0.0)
    cnt_acc[...] += jnp.sum(oh, axis=1, keepdims=True)
    for c in range(NCHUNK):
        mc = m_ref[c]
        sc = s_ref[c, 0] + s_ref[c, 1] + mc
        hc = jnp.maximum(sc * dinv + b_ref[:, c * CW:(c + 1) * CW], 0.0)
        pooled_acc[c] += jnp.dot(oh, hc, preferred_element_type=jnp.float32)

    @pl.when(i == GRID_N - 1)
    def _():
        cnt = jnp.maximum(cnt_acc[...], 1.0)
        logits = jnp.zeros((G, D_OUT), jnp.float32)
        for c in range(NCHUNK):
            logits = logits + jnp.dot(pooled_acc[c] / cnt,
                                      wfc_ref[c * CW:(c + 1) * CW, :],
                                      preferred_element_type=jnp.float32)
        logits = logits + bfc_ref[...]
        mx = jnp.max(logits, axis=1, keepdims=True)
        lse = jnp.log(jnp.sum(jnp.exp(logits - mx), axis=1, keepdims=True)) + mx
        out_ref[...] = logits - lse


_fin_call = pl.pallas_call(
    _fin_body,
    grid=(GRID_N,),
    in_specs=[
        pl.BlockSpec((NCHUNK, NC, TILE_N, CW), lambda i: (0, 0, i, 0)),
        pl.BlockSpec((NCHUNK, TILE_N, CW), lambda i: (0, i, 0)),
        pl.BlockSpec((TILE_N, 1), lambda i: (i, 0)),
        pl.BlockSpec((1, D_H), lambda i: (0, 0)),
        pl.BlockSpec((1, TILE_N), lambda i: (0, i)),
        pl.BlockSpec((D_H, D_OUT), lambda i: (0, 0)),
        pl.BlockSpec((1, D_OUT), lambda i: (0, 0)),
    ],
    out_specs=pl.BlockSpec((G, D_OUT), lambda i: (0, 0)),
    out_shape=jax.ShapeDtypeStruct((G, D_OUT), jnp.float32),
    scratch_shapes=[
        pltpu.VMEM((NCHUNK, G, CW), jnp.float32),
        pltpu.VMEM((G, 1), jnp.float32),
    ],
)


def kernel(x, edge_index, batch, W1, b1, W2, b2, Wfc, bfc):
    src = edge_index[0]
    dst = edge_index[1]
    pe = E_PAD - E
    srcr = jnp.concatenate([src, jnp.zeros((pe,), jnp.int32)]).reshape(E_PAD // EB, EB)
    dstr = jnp.concatenate([dst, jnp.full((pe,), N, jnp.int32)]).reshape(E_PAD // EB, EB)
    xp = jnp.pad(x, ((0, N_PAD - N), (0, 0)))
    batch_p = jnp.concatenate([batch, jnp.full((N_PAD - N,), G, jnp.int32)]).reshape(1, N_PAD)

    degp = _deg_call()(dstr)
    deg = degp.reshape(NC, N_PAD).sum(axis=0) + 1.0
    dinv = lax.rsqrt(deg).reshape(N_PAD, 1)

    spmm = _spmm_call()
    m1 = _mm1_call(xp, W1, dinv)
    s1 = spmm(m1, srcr, dstr).reshape(NCHUNK, NC, N_PAD, CW)
    m2 = _mm2_call(s1, m1, dinv, b1.reshape(1, D_H), W2)
    s2 = spmm(m2, srcr, dstr).reshape(NCHUNK, NC, N_PAD, CW)
    return _fin_call(s2, m2, dinv, b2.reshape(1, D_H), batch_p, Wfc,
                     bfc.reshape(1, D_OUT))


# speedup vs baseline: 4.9092x; 1.1871x over previous
import functools

import jax
import jax.numpy as jnp
from jax import lax
from jax.experimental import pallas as pl
from jax.experimental.pallas import tpu as pltpu
import jax.experimental.pallas.tpu_sc as plsc

N = 10000
E = 160000
G = 128
D_IN = 256
D_H = 512
D_OUT = 16

NC = 2
NS = 16
NW = NC * NS
L = 16

CW = 128
NCHUNK = D_H // CW
N_PAD = 10240
STRIPE = N_PAD // NS
EB = 64
NB = 80
E_PAD = NW * NB * EB
NB0 = 128
NB1 = 32
E_ROWS = E_PAD // EB
E_ROWS_ARR = E_ROWS + (NB0 - NB1)
TILE_N = 1280
GRID_N = N_PAD // TILE_N


def _deg_body(dst_hbm, out_hbm, idx_buf, ones_buf, zstripe, acc_sh):
    core = lax.axis_index("c")
    sid = lax.axis_index("s")
    wid = sid * NC + core
    one16 = jnp.ones((L,), jnp.float32)
    zero16 = jnp.zeros((L,), jnp.float32)

    def fill_ones(t, _):
        ones_buf[pl.ds(t * L, L)] = one16
        return 0
    lax.fori_loop(0, EB // L, fill_ones, 0)

    def fill_z(t, _):
        zstripe[pl.ds(t * L, L)] = zero16
        return 0
    lax.fori_loop(0, STRIPE // L, fill_z, 0)

    pltpu.sync_copy(zstripe, acc_sh.at[pl.ds(sid * STRIPE, STRIPE)])
    pltpu.sync_copy(dst_hbm.at[pl.ds(wid * NB, NB), :], idx_buf)
    plsc.subcore_barrier()

    def add_batch(j, _):
        pltpu.sync_copy(ones_buf, acc_sh.at[idx_buf.at[j]], add=True)
        return 0
    lax.fori_loop(0, NB, add_batch, 0)
    plsc.subcore_barrier()

    pltpu.sync_copy(acc_sh.at[pl.ds(sid * STRIPE, STRIPE)],
                    out_hbm.at[pl.ds(core * N_PAD + sid * STRIPE, STRIPE)])


@functools.cache
def _deg_call():
    mesh = plsc.VectorSubcoreMesh(core_axis_name="c", subcore_axis_name="s",
                                  num_cores=NC, num_subcores=NS)
    return pl.kernel(
        _deg_body,
        out_type=jax.ShapeDtypeStruct((NC * N_PAD,), jnp.float32),
        mesh=mesh,
        scratch_types=[
            pltpu.VMEM((NB, EB), jnp.int32),
            pltpu.VMEM((EB,), jnp.float32),
            pltpu.VMEM((STRIPE,), jnp.float32),
            pltpu.VMEM_SHARED((N_PAD,), jnp.float32),
        ],
    )


def _spmm_body(mp_hbm, src_hbm, dst_hbm, out_hbm,
               src_buf, dst_buf, rows_a, rows_b, sem_a, sem_b, acc_sh):
    core = lax.axis_index("c")
    sid = lax.axis_index("s")
    zero16 = jnp.zeros((L,), jnp.float32)

    nbt = jnp.where(core == 0, NB0, NB1)
    base = jnp.where(core == 0, sid * NB0, NS * NB0 + sid * NB1)
    pltpu.sync_copy(src_hbm.at[pl.ds(base, NB0), :], src_buf)
    pltpu.sync_copy(dst_hbm.at[pl.ds(base, NB0), :], dst_buf)

    for c in range(NCHUNK):
        def fill_z(t, _):
            r = t // (CW // L)
            k = (t % (CW // L)) * L
            rows_a[r, pl.ds(k, L)] = zero16
            return 0
        lax.fori_loop(0, EB * CW // L, fill_z, 0)
        for r in range(STRIPE // EB):
            pltpu.sync_copy(rows_a, acc_sh.at[pl.ds(sid * STRIPE + r * EB, EB), :])
        plsc.subcore_barrier()
        table = mp_hbm.at[c]
        pltpu.async_copy(table.at[src_buf.at[0]], rows_a, sem_a)
        pltpu.async_copy(table.at[src_buf.at[1]], rows_b, sem_b)

        def step(t, _):
            for b, (rbuf, sem) in enumerate(((rows_a, sem_a), (rows_b, sem_b))):
                j = 2 * t + b
                pltpu.make_async_copy(table.at[src_buf.at[j]], rbuf, sem).wait()
                pltpu.sync_copy(rbuf, acc_sh.at[dst_buf.at[j]], add=True)
                jj = j + 2

                @pl.when(jj < nbt)
                def _():
                    pltpu.async_copy(table.at[src_buf.at[jj]], rbuf, sem)
            return 0
        lax.fori_loop(0, nbt // 2, step, 0)
        plsc.subcore_barrier()
        pltpu.sync_copy(
            acc_sh.at[pl.ds(sid * STRIPE, STRIPE), :],
            out_hbm.at[c].at[pl.ds(core * N_PAD + sid * STRIPE, STRIPE), :])
        plsc.subcore_barrier()


@functools.cache
def _spmm_call():
    mesh = plsc.VectorSubcoreMesh(core_axis_name="c", subcore_axis_name="s",
                                  num_cores=NC, num_subcores=NS)
    return pl.kernel(
        _spmm_body,
        out_type=jax.ShapeDtypeStruct((NCHUNK, NC * N_PAD, CW), jnp.float32),
        mesh=mesh,
        scratch_types=[
            pltpu.VMEM((NB0, EB), jnp.int32),
            pltpu.VMEM((NB0, EB), jnp.int32),
            pltpu.VMEM((EB, CW), jnp.float32),
            pltpu.VMEM((EB, CW), jnp.float32),
            pltpu.SemaphoreType.DMA,
            pltpu.SemaphoreType.DMA,
            pltpu.VMEM_SHARED((N_PAD, CW), jnp.float32),
        ],
    )


def _mm1_body(x_ref, w_ref, dinv_ref, out_ref):
    m = jnp.dot(x_ref[...], w_ref[...], preferred_element_type=jnp.float32)
    m = m * dinv_ref[...]
    for c in range(NCHUNK):
        out_ref[c] = m[:, c * CW:(c + 1) * CW]


_mm1_call = pl.pallas_call(
    _mm1_body,
    grid=(GRID_N,),
    in_specs=[
        pl.BlockSpec((TILE_N, D_IN), lambda i: (i, 0)),
        pl.BlockSpec((D_IN, D_H), lambda i: (0, 0)),
        pl.BlockSpec((TILE_N, 1), lambda i: (i, 0)),
    ],
    out_specs=pl.BlockSpec((NCHUNK, TILE_N, CW), lambda i: (0, i, 0)),
    out_shape=jax.ShapeDtypeStruct((NCHUNK, N_PAD, CW), jnp.float32),
)


def _mm2_body(s_ref, m_ref, dinv_ref, b_ref, w_ref, out_ref):
    dinv = dinv_ref[...]
    acc = jnp.zeros((TILE_N, D_H), jnp.float32)
    for c in range(NCHUNK):
        mc = m_ref[c]
        sc = s_ref[c, 0] + s_ref[c, 1] + mc
        hc = jnp.maximum(sc * dinv + b_ref[:, c * CW:(c + 1) * CW], 0.0)
        acc = acc + jnp.dot(hc, w_ref[c * CW:(c + 1) * CW, :],
                            preferred_element_type=jnp.float32)
    acc = acc * dinv
    for c in range(NCHUNK):
        out_ref[c] = acc[:, c * CW:(c + 1) * CW]


_mm2_call = pl.pallas_call(
    _mm2_body,
    grid=(GRID_N,),
    in_specs=[
        pl.BlockSpec((NCHUNK, NC, TILE_N, CW), lambda i: (0, 0, i, 0)),
        pl.BlockSpec((NCHUNK, TILE_N, CW), lambda i: (0, i, 0)),
        pl.BlockSpec((TILE_N, 1), lambda i: (i, 0)),
        pl.BlockSpec((1, D_H), lambda i: (0, 0)),
        pl.BlockSpec((D_H, D_H), lambda i: (0, 0)),
    ],
    out_specs=pl.BlockSpec((NCHUNK, TILE_N, CW), lambda i: (0, i, 0)),
    out_shape=jax.ShapeDtypeStruct((NCHUNK, N_PAD, CW), jnp.float32),
)


def _fin_body(s_ref, m_ref, dinv_ref, b_ref, batch_ref, wfc_ref, bfc_ref,
              out_ref, pooled_acc, cnt_acc):
    i = pl.program_id(0)

    @pl.when(i == 0)
    def _():
        pooled_acc[...] = jnp.zeros_like(pooled_acc)
        cnt_acc[...] = jnp.zeros_like(cnt_acc)

    dinv = dinv_ref[...]
    bt = batch_ref[...]
    gi = lax.broadcasted_iota(jnp.int32, (G, TILE_N), 0)
    oh = jnp.where(gi == bt, 1.0, 0.0)
    cnt_acc[...] += jnp.sum(oh, axis=1, keepdims=True)
    for c in range(NCHUNK):
        mc = m_ref[c]
        sc = s_ref[c, 0] + s_ref[c, 1] + mc
        hc = jnp.maximum(sc * dinv + b_ref[:, c * CW:(c + 1) * CW], 0.0)
        pooled_acc[c] += jnp.dot(oh, hc, preferred_element_type=jnp.float32)

    @pl.when(i == GRID_N - 1)
    def _():
        cnt = jnp.maximum(cnt_acc[...], 1.0)
        logits = jnp.zeros((G, D_OUT), jnp.float32)
        for c in range(NCHUNK):
            logits = logits + jnp.dot(pooled_acc[c] / cnt,
                                      wfc_ref[c * CW:(c + 1) * CW, :],
                                      preferred_element_type=jnp.float32)
        logits = logits + bfc_ref[...]
        mx = jnp.max(logits, axis=1, keepdims=True)
        lse = jnp.log(jnp.sum(jnp.exp(logits - mx), axis=1, keepdims=True)) + mx
        out_ref[...] = logits - lse


_fin_call = pl.pallas_call(
    _fin_body,
    grid=(GRID_N,),
    in_specs=[
        pl.BlockSpec((NCHUNK, NC, TILE_N, CW), lambda i: (0, 0, i, 0)),
        pl.BlockSpec((NCHUNK, TILE_N, CW), lambda i: (0, i, 0)),
        pl.BlockSpec((TILE_N, 1), lambda i: (i, 0)),
        pl.BlockSpec((1, D_H), lambda i: (0, 0)),
        pl.BlockSpec((1, TILE_N), lambda i: (0, i)),
        pl.BlockSpec((D_H, D_OUT), lambda i: (0, 0)),
        pl.BlockSpec((1, D_OUT), lambda i: (0, 0)),
    ],
    out_specs=pl.BlockSpec((G, D_OUT), lambda i: (0, 0)),
    out_shape=jax.ShapeDtypeStruct((G, D_OUT), jnp.float32),
    scratch_shapes=[
        pltpu.VMEM((NCHUNK, G, CW), jnp.float32),
        pltpu.VMEM((G, 1), jnp.float32),
    ],
)


def kernel(x, edge_index, batch, W1, b1, W2, b2, Wfc, bfc):
    src = edge_index[0]
    dst = edge_index[1]
    pe = E_ROWS_ARR * EB - E
    srcr = jnp.concatenate([src, jnp.zeros((pe,), jnp.int32)]).reshape(E_ROWS_ARR, EB)
    dstr = jnp.concatenate([dst, jnp.full((pe,), N, jnp.int32)]).reshape(E_ROWS_ARR, EB)
    xp = jnp.pad(x, ((0, N_PAD - N), (0, 0)))
    batch_p = jnp.concatenate([batch, jnp.full((N_PAD - N,), G, jnp.int32)]).reshape(1, N_PAD)

    degp = _deg_call()(dstr)
    deg = degp.reshape(NC, N_PAD).sum(axis=0) + 1.0
    dinv = lax.rsqrt(deg).reshape(N_PAD, 1)

    spmm = _spmm_call()
    m1 = _mm1_call(xp, W1, dinv)
    s1 = spmm(m1, srcr, dstr).reshape(NCHUNK, NC, N_PAD, CW)
    m2 = _mm2_call(s1, m1, dinv, b1.reshape(1, D_H), W2)
    s2 = spmm(m2, srcr, dstr).reshape(NCHUNK, NC, N_PAD, CW)
    return _fin_call(s2, m2, dinv, b2.reshape(1, D_H), batch_p, Wfc,
                     bfc.reshape(1, D_OUT))
